# Initial kernel scaffold; baseline (speedup 1.0000x reference)
import jax
import jax.numpy as jnp
from jax.experimental import pallas as pl


def kernel(node_hidden, edge_hidden, edge_index, W, b):
    raise NotImplementedError("write your pallas kernel here")


# speedup vs baseline: 4.5020x; 4.5020x over previous
import functools

import jax
import jax.numpy as jnp
from jax import lax
from jax.experimental import pallas as pl
from jax.experimental.pallas import tpu as pltpu
from jax.experimental.pallas import tpu_sc as plsc

N = 10000
E = 320000
D = 128

_NC = 2
_NS = 16
_NW = _NC * _NS
_EPW = E // _NW
_L = 16


def _nodeproj_body(x_ref, w_ref, o_ref):
    o_ref[...] = jnp.dot(x_ref[...], w_ref[...], preferred_element_type=jnp.float32)


def _node_projections(node_hidden, w12):
    return pl.pallas_call(
        _nodeproj_body,
        out_shape=jax.ShapeDtypeStruct((N, 2), jnp.float32),
    )(node_hidden, w12)


_sc_mesh = plsc.VectorSubcoreMesh(
    core_axis_name="c", subcore_axis_name="s", num_cores=_NC, num_subcores=_NS
)


@functools.partial(
    pl.kernel,
    out_type=jax.ShapeDtypeStruct((E,), jnp.float32),
    mesh=_sc_mesh,
    compiler_params=pltpu.CompilerParams(needs_layout_passes=False),
    scratch_types=[
        pltpu.VMEM((2 * N,), jnp.float32),
        pltpu.VMEM((_EPW,), jnp.int32),
        pltpu.VMEM((_EPW,), jnp.int32),
        pltpu.VMEM((_EPW,), jnp.float32),
    ],
)
def _sc_gather(st_hbm, src_hbm, dst_hbm, out_hbm, st_v, src_v, dst_v, g_v):
    wid = lax.axis_index("s") * _NC + lax.axis_index("c")
    base = wid * _EPW
    pltpu.sync_copy(st_hbm, st_v)
    pltpu.sync_copy(src_hbm.at[pl.ds(base, _EPW)], src_v)
    pltpu.sync_copy(dst_hbm.at[pl.ds(base, _EPW)], dst_v)

    def body(i, carry):
        sl = pl.ds(i * _L, _L)
        si = src_v[sl]
        di = dst_v[sl]
        g = plsc.load_gather(st_v, [si * 2]) + plsc.load_gather(st_v, [di * 2 + 1])
        g_v[sl] = g
        return carry

    lax.fori_loop(0, _EPW // _L, body, 0)
    pltpu.sync_copy(g_v, out_hbm.at[pl.ds(base, _EPW)])


_BE = 6400


def _decode_body(eh_ref, g_ref, w_ref, b_ref, o_ref):
    acc = jnp.dot(eh_ref[...], w_ref[...], preferred_element_type=jnp.float32)
    o_ref[...] = acc + g_ref[...] + b_ref[0, 0]


def _edge_decode(edge_hidden, g, w0, b):
    return pl.pallas_call(
        _decode_body,
        grid=(E // _BE,),
        in_specs=[
            pl.BlockSpec((_BE, D), lambda i: (i, 0)),
            pl.BlockSpec((_BE, 1), lambda i: (i, 0)),
            pl.BlockSpec((D, 1), lambda i: (0, 0)),
            pl.BlockSpec(memory_space=pltpu.SMEM),
        ],
        out_specs=pl.BlockSpec((_BE, 1), lambda i: (i, 0)),
        out_shape=jax.ShapeDtypeStruct((E, 1), jnp.float32),
    )(edge_hidden, g, w0, b)


def kernel(node_hidden, edge_hidden, edge_index, W, b):
    src = edge_index[0].astype(jnp.int32)
    dst = edge_index[1].astype(jnp.int32)
    w0 = W[:D]
    w12 = jnp.concatenate([W[D : 2 * D], W[2 * D :]], axis=1)

    st = _node_projections(node_hidden, w12).reshape(2 * N)
    g = _sc_gather(st, src, dst).reshape(E, 1)
    return _edge_decode(edge_hidden, g, w0, b.reshape(1, 1))
